# baseline (speedup 1.0000x reference)
import functools

import jax
import jax.numpy as jnp
from jax import lax
from jax.experimental import pallas as pl
from jax.experimental.pallas import tpu as pltpu
from jax.experimental.pallas import tpu_sc as plsc


def _lookup_kernel(NI, NJ, D, CH):
    info = plsc.get_sparse_core_info()
    NC, NS = info.num_cores, info.num_subcores
    NW = NC * NS
    NB_I = NI // CH
    per_w = NB_I // NW
    n_sub = per_w * NJ
    W = CH * NJ
    PITCH = CH + 1
    assert NI % CH == 0 and NB_I % NW == 0 and NJ % 2 == 0
    mesh = plsc.VectorSubcoreMesh(core_axis_name="c", subcore_axis_name="s")

    @functools.partial(
        pl.kernel,
        out_type=jax.ShapeDtypeStruct((NJ, D // 8, NB_I, 8, CH), jnp.float32),
        mesh=mesh,
        scratch_types=[
            pltpu.VMEM((W,), jnp.int32),
            pltpu.VMEM((CH,), jnp.int32),
            pltpu.VMEM((CH,), jnp.int32),
            pltpu.VMEM((CH,), jnp.int32),
            pltpu.VMEM((CH,), jnp.int32),
            pltpu.VMEM((4, CH, D), jnp.float32),
            pltpu.VMEM((2, D, PITCH), jnp.float32),
            pltpu.SemaphoreType.DMA,
            pltpu.SemaphoreType.DMA,
            pltpu.SemaphoreType.DMA,
            pltpu.SemaphoreType.DMA,
            pltpu.SemaphoreType.DMA,
            pltpu.SemaphoreType.DMA,
        ],
        compiler_params=pltpu.CompilerParams(
            use_tc_tiling_on_sc=False, needs_layout_passes=False),
    )
    def k(idx1d, t64, out, win, p0, p1, p2, p3, grows, oblk,
          g0, g1, g2, g3, s0, s1):
        wid = lax.axis_index("s") * NC + lax.axis_index("c")
        prefs = (p0, p1, p2, p3)
        gsems = (g0, g1, g2, g3)
        ssems = (s0, s1)
        lanes = lax.iota(jnp.int32, 16)
        lanesj = lanes * NJ

        def decode(n):
            m = n // NJ
            j = n - m * NJ
            return j, wid + m * NW

        def prep(n, s):
            j, ib = decode(n)

            @pl.when(j == 0)
            def _():
                pltpu.sync_copy(idx1d.at[pl.ds(ib * W, W)], win)

            pref = prefs[s]
            for g in range(CH // 16):
                v = plsc.load_gather(win, [lanesj + (g * 16 * NJ + j)])
                pref[pl.ds(g * 16, 16)] = v
            pltpu.async_copy(t64.at[pref], grows.at[s], gsems[s])

        def gather_wait(s):
            pltpu.make_async_copy(
                t64.at[prefs[s]], grows.at[s], gsems[s]).wait()

        def transpose(s, so):
            gref = grows.at[s]
            oref = oblk.at[so]

            @pl.loop(0, CH // 4)
            def _(kq):
                k0 = kq * 4
                ksplat0 = jnp.full((16,), k0, jnp.int32)
                for r in range(4):
                    ksplat = ksplat0 + r
                    for c in range(D // 16):
                        v = gref[k0 + r, pl.ds(c * 16, 16)]
                        plsc.store_scatter(oref, [lanes + c * 16, ksplat], v)

        def store_start(n, s):
            j, ib = decode(n)
            for tr in range(D // 8):
                pltpu.make_async_copy(
                    oblk.at[s, pl.ds(8 * tr, 8), pl.ds(0, CH)],
                    out.at[j, tr, ib], ssems[s]).start()

        def store_wait(s):
            for tr in range(D // 8):
                pltpu.make_async_copy(
                    oblk.at[s, pl.ds(8 * tr, 8), pl.ds(0, CH)],
                    out.at[0, tr, 0], ssems[s]).wait()

        prep(0, 0)
        prep(1, 1)
        prep(2, 2)

        @pl.loop(0, n_sub // 4)
        def _(q):
            for r in range(4):
                n = 4 * q + r
                so = r & 1
                gather_wait(r)

                @pl.when(n >= 2)
                def _():
                    store_wait(so)

                transpose(r, so)
                store_start(n, so)

                @pl.when(n + 3 < n_sub)
                def _():
                    prep(n + 3, (r + 3) % 4)

        store_wait(0)
        store_wait(1)

    return k


@jax.jit
def kernel(data, table):
    NI, NJ = data.shape
    V, D = table.shape
    idx1d = data.reshape(NI * NJ)
    CH = 128
    out_t = _lookup_kernel(NI, NJ, D, CH)(idx1d, table)
    out_phys = out_t.transpose(0, 1, 3, 2, 4).reshape(NJ, D, NI)
    return out_phys.transpose(2, 0, 1)


# speedup vs baseline: 1.0012x; 1.0012x over previous
import functools

import jax
import jax.numpy as jnp
from jax import lax
from jax.experimental import pallas as pl
from jax.experimental.pallas import tpu as pltpu
from jax.experimental.pallas import tpu_sc as plsc


def _lookup_kernel(NI, NJ, D, CH):
    info = plsc.get_sparse_core_info()
    NC, NS = info.num_cores, info.num_subcores
    NW = NC * NS
    NB_I = NI // CH
    per_w = NB_I // NW
    n_sub = per_w * NJ
    W = CH * NJ
    PITCH = CH + 1
    assert NI % CH == 0 and NB_I % NW == 0 and n_sub % 4 == 0 and D % 16 == 0
    mesh = plsc.VectorSubcoreMesh(core_axis_name="c", subcore_axis_name="s")

    @functools.partial(
        pl.kernel,
        out_type=jax.ShapeDtypeStruct((NJ, D // 8, NB_I, 8, CH), jnp.float32),
        mesh=mesh,
        scratch_types=[
            pltpu.VMEM((W,), jnp.int32),
            pltpu.VMEM((CH,), jnp.int32),
            pltpu.VMEM((CH,), jnp.int32),
            pltpu.VMEM((CH,), jnp.int32),
            pltpu.VMEM((CH,), jnp.int32),
            pltpu.VMEM((4, CH, D), jnp.float32),
            pltpu.VMEM((2, D, PITCH), jnp.float32),
            pltpu.SemaphoreType.DMA,
            pltpu.SemaphoreType.DMA,
            pltpu.SemaphoreType.DMA,
            pltpu.SemaphoreType.DMA,
            pltpu.SemaphoreType.DMA,
            pltpu.SemaphoreType.DMA,
        ],
        compiler_params=pltpu.CompilerParams(
            use_tc_tiling_on_sc=False, needs_layout_passes=False),
    )
    def k(idx1d, t64, out, win, p0, p1, p2, p3, grows, oblk,
          g0, g1, g2, g3, s0, s1):
        wid = lax.axis_index("s") * NC + lax.axis_index("c")
        prefs = (p0, p1, p2, p3)
        gsems = (g0, g1, g2, g3)
        ssems = (s0, s1)
        lanes = lax.iota(jnp.int32, 16)
        lanesj = lanes * NJ

        def decode(n):
            m = n // NJ
            j = n - m * NJ
            return j, wid + m * NW

        def prep(n, s):
            j, ib = decode(n)

            @pl.when(j == 0)
            def _():
                pltpu.sync_copy(idx1d.at[pl.ds(ib * W, W)], win)

            pref = prefs[s]
            for g in range(CH // 16):
                v = plsc.load_gather(win, [lanesj + (g * 16 * NJ + j)])
                pref[pl.ds(g * 16, 16)] = v
            pltpu.async_copy(t64.at[pref], grows.at[s], gsems[s])

        def gather_wait(s):
            pltpu.make_async_copy(
                t64.at[prefs[s]], grows.at[s], gsems[s]).wait()

        def transpose(s, so):
            gref = grows.at[s]
            oref = oblk.at[so]

            @pl.loop(0, CH // 4)
            def _(kq):
                k0 = kq * 4
                ksplat0 = jnp.full((16,), k0, jnp.int32)
                for r in range(4):
                    ksplat = ksplat0 + r
                    for c in range(D // 16):
                        v = gref[k0 + r, pl.ds(c * 16, 16)]
                        plsc.store_scatter(oref, [lanes + c * 16, ksplat], v)

        def store_start(n, s):
            j, ib = decode(n)
            for tr in range(D // 8):
                pltpu.make_async_copy(
                    oblk.at[s, pl.ds(8 * tr, 8), pl.ds(0, CH)],
                    out.at[j, tr, ib], ssems[s]).start()

        def store_wait(s):
            for tr in range(D // 8):
                pltpu.make_async_copy(
                    oblk.at[s, pl.ds(8 * tr, 8), pl.ds(0, CH)],
                    out.at[0, tr, 0], ssems[s]).wait()

        prep(0, 0)
        prep(1, 1)
        prep(2, 2)

        @pl.loop(0, n_sub // 4)
        def _(q):
            for r in range(4):
                n = 4 * q + r
                so = r & 1
                gather_wait(r)

                @pl.when(n >= 2)
                def _():
                    store_wait(so)

                transpose(r, so)
                store_start(n, so)

                @pl.when(n + 3 < n_sub)
                def _():
                    prep(n + 3, (r + 3) % 4)

        store_wait(0)
        store_wait(1)

    return k


@jax.jit
def kernel(data, table):
    NI, NJ = data.shape
    V, D = table.shape
    idx1d = data.reshape(NI * NJ)
    CH = 128
    out_t = _lookup_kernel(NI, NJ, D, CH)(idx1d, table)
    out_phys = out_t.transpose(0, 1, 3, 2, 4).reshape(NJ, D, NI)
    return out_phys.transpose(2, 0, 1)
